# baseline (speedup 1.0000x reference)
import functools
import math

import jax
import jax.numpy as jnp
from jax.experimental import pallas as pl
from jax.experimental.pallas import tpu as pltpu

TOP_K = 8
N_EXP = 64
EVAL_CAPACITY = 1.25
MIN_CAPACITY = 4

BN = 256


def _capacity(num_tokens: int) -> int:
    capacity = math.floor(TOP_K * EVAL_CAPACITY * num_tokens / N_EXP)
    capacity += capacity % 2
    capacity = max(capacity, MIN_CAPACITY)
    return int(capacity)


def _body(capacity, nblocks,
          x_ref, wg_ref, u_ref, jt_ref,
          mask_ref, pmask_ref, idxo_ref, rank_ref,
          idx_s, probs_s, rloc_s, hist_s):
    i = pl.program_id(0)
    G = nblocks
    KE = TOP_K * N_EXP

    @pl.when(i < G)
    def _phase1():
        xb = x_ref[...]
        wg = wg_ref[...]
        logitsT = jax.lax.dot_general(
            wg, xb, (((1,), (1,)), ((), ())),
            preferred_element_type=jnp.float32,
        )

        iota_sub = jax.lax.broadcasted_iota(jnp.int32, (N_EXP, BN), 0)
        work = logitsT
        idx_rows = []
        val_rows = []
        for _ in range(TOP_K):
            m = jnp.max(work, axis=0, keepdims=True)
            sel = jnp.where(work == m, iota_sub, N_EXP)
            ij = jnp.min(sel, axis=0, keepdims=True)
            idx_rows.append(ij)
            val_rows.append(m)
            work = jnp.where(iota_sub == ij, -jnp.inf, work)
        idxT = jnp.concatenate(idx_rows, axis=0)
        tvT = jnp.concatenate(val_rows, axis=0)

        e = jnp.exp(tvT - tvT[0:1, :])
        probsT = e / jnp.sum(e, axis=0, keepdims=True)

        ohT = jnp.concatenate(
            [(idxT[j : j + 1, :] == iota_sub).astype(jnp.float32)
             for j in range(TOP_K)],
            axis=0,
        )
        csumT = jax.lax.dot_general(
            ohT, u_ref[...], (((1,), (0,)), ((), ())),
            preferred_element_type=jnp.float32,
        )
        rloc_rows = []
        for j in range(TOP_K):
            sl = slice(j * N_EXP, (j + 1) * N_EXP)
            rloc_rows.append(
                jnp.sum(ohT[sl, :] * csumT[sl, :], axis=0, keepdims=True))
        rlocT = jnp.concatenate(rloc_rows, axis=0) - 1.0

        ones_row = jnp.ones((1, BN), jnp.float32)
        hist_row = jax.lax.dot_general(
            ones_row, ohT, (((1,), (1,)), ((), ())),
            preferred_element_type=jnp.float32,
        )

        idx_s[pl.ds(i * TOP_K, TOP_K), :] = idxT
        probs_s[pl.ds(i * TOP_K, TOP_K), :] = probsT
        rloc_s[pl.ds(i * TOP_K, TOP_K), :] = rlocT
        hist_s[pl.ds(i, 1), :] = hist_row

    @pl.when(i >= G)
    def _phase2():
        b = i - G
        idxT = idx_s[pl.ds(b * TOP_K, TOP_K), :]
        probsT = probs_s[pl.ds(b * TOP_K, TOP_K), :]
        rlocT = rloc_s[pl.ds(b * TOP_K, TOP_K), :]
        hist = hist_s[...]

        gi = jax.lax.broadcasted_iota(jnp.int32, (G, 2), 0)
        ci = jax.lax.broadcasted_iota(jnp.int32, (G, 2), 1)
        selm = jnp.where(ci == 0, (gi < b).astype(jnp.float32), 1.0)
        bt = jax.lax.dot_general(
            hist, selm, (((0,), (0,)), ((), ())),
            preferred_element_type=jnp.float32,
        )
        bexcl = bt[:, 0:1]
        tot = bt[:, 1:2]

        iota_sub = jax.lax.broadcasted_iota(jnp.int32, (N_EXP, BN), 0)
        acc = jnp.zeros((N_EXP, 1), jnp.float32)
        rank_rows = []
        keep_rows = []
        for j in range(TOP_K):
            sl = slice(j * N_EXP, (j + 1) * N_EXP)
            base_j = acc + bexcl[sl, :]
            mask_j = idxT[j : j + 1, :] == iota_sub
            contrib = jnp.sum(jnp.where(mask_j, base_j, 0.0),
                              axis=0, keepdims=True)
            rank_j = rlocT[j : j + 1, :] + contrib
            keep_j = (rank_j < float(capacity)).astype(jnp.float32)
            rank_rows.append(rank_j)
            keep_rows.append(keep_j)
            acc = acc + tot[sl, :]
        rankT = jnp.concatenate(rank_rows, axis=0)
        keepT = jnp.concatenate(keep_rows, axis=0)

        stack = jnp.concatenate(
            [idxT.astype(jnp.float32), probsT, rankT, keepT], axis=0)
        st = stack.T
        idx_tok = st[:, 0 : TOP_K]
        probs_tok = st[:, TOP_K : 2 * TOP_K]
        rank_tok = st[:, 2 * TOP_K : 3 * TOP_K]
        keep_tok = st[:, 3 * TOP_K : 4 * TOP_K]

        pmask_ref[...] = probs_tok * keep_tok
        idxo_ref[...] = idx_tok.astype(jnp.int32)
        rank_ref[...] = rank_tok.astype(jnp.int32)

        ce = jax.lax.dot_general(
            idx_tok, jt_ref[...], (((1,), (0,)), ((), ())),
            preferred_element_type=jnp.float32,
        )
        keep_exp = jax.lax.dot_general(
            keep_tok, jt_ref[...], (((1,), (0,)), ((), ())),
            preferred_element_type=jnp.float32,
        )
        erow = (jax.lax.broadcasted_iota(jnp.int32, (1, KE), 1)
                & (N_EXP - 1)).astype(jnp.float32)
        maskm = jnp.where(ce == erow, keep_exp, 0.0)
        for j in range(TOP_K):
            sl = slice(j * N_EXP, (j + 1) * N_EXP)
            mask_ref[:, j, :] = maskm[:, sl].astype(jnp.int32)


def kernel(x, w_g):
    B, T, C = x.shape
    num_tokens = B * T
    x_flat = x.reshape(num_tokens, C)
    G = num_tokens // BN
    capacity = _capacity(num_tokens)
    KE = TOP_K * N_EXP

    r_i = jax.lax.broadcasted_iota(jnp.int32, (BN, BN), 0)
    c_i = jax.lax.broadcasted_iota(jnp.int32, (BN, BN), 1)
    u_incl = (r_i <= c_i).astype(jnp.float32)
    jr = jax.lax.broadcasted_iota(jnp.int32, (TOP_K, KE), 0)
    jc = jax.lax.broadcasted_iota(jnp.int32, (TOP_K, KE), 1)
    jt = (jr == jc // N_EXP).astype(jnp.float32)

    mask, pmask, idxo, rank = pl.pallas_call(
        functools.partial(_body, capacity, G),
        grid=(2 * G,),
        in_specs=[
            pl.BlockSpec((BN, C), lambda i: (jnp.minimum(i, G - 1), 0)),
            pl.BlockSpec((N_EXP, C), lambda i: (0, 0)),
            pl.BlockSpec((BN, BN), lambda i: (0, 0)),
            pl.BlockSpec((TOP_K, KE), lambda i: (0, 0)),
        ],
        out_specs=[
            pl.BlockSpec((BN, TOP_K, N_EXP),
                         lambda i: (jnp.maximum(i - G, 0), 0, 0)),
            pl.BlockSpec((BN, TOP_K), lambda i: (jnp.maximum(i - G, 0), 0)),
            pl.BlockSpec((BN, TOP_K), lambda i: (jnp.maximum(i - G, 0), 0)),
            pl.BlockSpec((BN, TOP_K), lambda i: (jnp.maximum(i - G, 0), 0)),
        ],
        out_shape=[
            jax.ShapeDtypeStruct((num_tokens, TOP_K, N_EXP), jnp.int32),
            jax.ShapeDtypeStruct((num_tokens, TOP_K), jnp.float32),
            jax.ShapeDtypeStruct((num_tokens, TOP_K), jnp.int32),
            jax.ShapeDtypeStruct((num_tokens, TOP_K), jnp.int32),
        ],
        scratch_shapes=[
            pltpu.VMEM((G * TOP_K, BN), jnp.int32),
            pltpu.VMEM((G * TOP_K, BN), jnp.float32),
            pltpu.VMEM((G * TOP_K, BN), jnp.float32),
            pltpu.VMEM((G, KE), jnp.float32),
        ],
    )(x_flat, w_g, u_incl, jt)

    return (mask, pmask, idxo, rank)


# speedup vs baseline: 1.4067x; 1.4067x over previous
import functools
import math

import jax
import jax.numpy as jnp
from jax.experimental import pallas as pl
from jax.experimental.pallas import tpu as pltpu

TOP_K = 8
N_EXP = 64
EVAL_CAPACITY = 1.25
MIN_CAPACITY = 4

BN = 256


def _capacity(num_tokens: int) -> int:
    capacity = math.floor(TOP_K * EVAL_CAPACITY * num_tokens / N_EXP)
    capacity += capacity % 2
    capacity = max(capacity, MIN_CAPACITY)
    return int(capacity)


def _body(capacity, nblocks,
          x_ref, wg_ref, u_ref, jt_ref,
          mask_ref, pmask_ref, idxo_ref, rank_ref,
          idx_s, probs_s, rloc_s, hist_s):
    i = pl.program_id(0)
    G = nblocks
    KE = TOP_K * N_EXP

    @pl.when(i < G)
    def _phase1():
        xb = x_ref[...]
        wg = wg_ref[...]
        logitsT = jax.lax.dot_general(
            wg, xb, (((1,), (1,)), ((), ())),
            preferred_element_type=jnp.float32,
        )

        iota_sub = jax.lax.broadcasted_iota(jnp.int32, (N_EXP, BN), 0)
        work = logitsT
        idx_rows = []
        val_rows = []
        for _ in range(TOP_K):
            m = jnp.max(work, axis=0, keepdims=True)
            sel = jnp.where(work == m, iota_sub, N_EXP)
            ij = jnp.min(sel, axis=0, keepdims=True)
            idx_rows.append(ij)
            val_rows.append(m)
            work = jnp.where(iota_sub == ij, -jnp.inf, work)
        idxT = jnp.concatenate(idx_rows, axis=0)
        tvT = jnp.concatenate(val_rows, axis=0)

        e = jnp.exp(tvT - tvT[0:1, :])
        probsT = e / jnp.sum(e, axis=0, keepdims=True)

        ohT = jnp.concatenate(
            [(idxT[j : j + 1, :] == iota_sub).astype(jnp.float32)
             for j in range(TOP_K)],
            axis=0,
        )
        csumT = jax.lax.dot_general(
            ohT, u_ref[...], (((1,), (0,)), ((), ())),
            preferred_element_type=jnp.float32,
        )
        rloc_rows = []
        for j in range(TOP_K):
            sl = slice(j * N_EXP, (j + 1) * N_EXP)
            rloc_rows.append(
                jnp.sum(ohT[sl, :] * csumT[sl, :], axis=0, keepdims=True))
        rlocT = jnp.concatenate(rloc_rows, axis=0) - 1.0

        ones_row = jnp.ones((1, BN), jnp.float32)
        hist_row = jax.lax.dot_general(
            ones_row, ohT, (((1,), (1,)), ((), ())),
            preferred_element_type=jnp.float32,
        )

        idx_s[pl.ds(i * TOP_K, TOP_K), :] = idxT
        probs_s[pl.ds(i * TOP_K, TOP_K), :] = probsT
        rloc_s[pl.ds(i * TOP_K, TOP_K), :] = rlocT
        hist_s[pl.ds(i, 1), :] = hist_row

    @pl.when(i >= G)
    def _phase2():
        b = i - G
        idxT = idx_s[pl.ds(b * TOP_K, TOP_K), :]
        probsT = probs_s[pl.ds(b * TOP_K, TOP_K), :]
        rlocT = rloc_s[pl.ds(b * TOP_K, TOP_K), :]
        hist = hist_s[...]

        gi = jax.lax.broadcasted_iota(jnp.int32, (G, 2), 0)
        ci = jax.lax.broadcasted_iota(jnp.int32, (G, 2), 1)
        selm = jnp.where(ci == 0, (gi < b).astype(jnp.float32), 1.0)
        bt = jax.lax.dot_general(
            hist, selm, (((0,), (0,)), ((), ())),
            preferred_element_type=jnp.float32,
        )
        bexcl = bt[:, 0:1]
        tot = bt[:, 1:2]

        iota_sub = jax.lax.broadcasted_iota(jnp.int32, (N_EXP, BN), 0)
        acc = jnp.zeros((N_EXP, 1), jnp.float32)
        rank_rows = []
        keep_rows = []
        for j in range(TOP_K):
            sl = slice(j * N_EXP, (j + 1) * N_EXP)
            base_j = acc + bexcl[sl, :]
            mask_j = idxT[j : j + 1, :] == iota_sub
            contrib = jnp.sum(jnp.where(mask_j, base_j, 0.0),
                              axis=0, keepdims=True)
            rank_j = rlocT[j : j + 1, :] + contrib
            keep_j = (rank_j < float(capacity)).astype(jnp.float32)
            rank_rows.append(rank_j)
            keep_rows.append(keep_j)
            acc = acc + tot[sl, :]
        rankT = jnp.concatenate(rank_rows, axis=0)
        keepT = jnp.concatenate(keep_rows, axis=0)

        stack = jnp.concatenate(
            [idxT.astype(jnp.float32), probsT, rankT, keepT], axis=0)
        st = stack.T
        idx_tok = st[:, 0 : TOP_K]
        probs_tok = st[:, TOP_K : 2 * TOP_K]
        rank_tok = st[:, 2 * TOP_K : 3 * TOP_K]
        keep_tok = st[:, 3 * TOP_K : 4 * TOP_K]

        pmask_ref[...] = probs_tok * keep_tok
        idxo_ref[...] = idx_tok.astype(jnp.int32)
        rank_ref[...] = rank_tok.astype(jnp.int32)

        ce = jax.lax.dot_general(
            idx_tok, jt_ref[...], (((1,), (0,)), ((), ())),
            preferred_element_type=jnp.float32,
        )
        keep_exp = jax.lax.dot_general(
            keep_tok, jt_ref[...], (((1,), (0,)), ((), ())),
            preferred_element_type=jnp.float32,
        )
        erow = (jax.lax.broadcasted_iota(jnp.int32, (1, KE), 1)
                & (N_EXP - 1)).astype(jnp.float32)
        maskm = jnp.where(ce == erow, keep_exp, 0.0)
        mask_ref[...] = maskm.astype(jnp.int32)


def kernel(x, w_g):
    B, T, C = x.shape
    num_tokens = B * T
    x_flat = x.reshape(num_tokens, C)
    G = num_tokens // BN
    capacity = _capacity(num_tokens)
    KE = TOP_K * N_EXP

    r_i = jax.lax.broadcasted_iota(jnp.int32, (BN, BN), 0)
    c_i = jax.lax.broadcasted_iota(jnp.int32, (BN, BN), 1)
    u_incl = (r_i <= c_i).astype(jnp.float32)
    jr = jax.lax.broadcasted_iota(jnp.int32, (TOP_K, KE), 0)
    jc = jax.lax.broadcasted_iota(jnp.int32, (TOP_K, KE), 1)
    jt = (jr == jc // N_EXP).astype(jnp.float32)

    mask, pmask, idxo, rank = pl.pallas_call(
        functools.partial(_body, capacity, G),
        grid=(2 * G,),
        in_specs=[
            pl.BlockSpec((BN, C), lambda i: (jnp.minimum(i, G - 1), 0)),
            pl.BlockSpec((N_EXP, C), lambda i: (0, 0)),
            pl.BlockSpec((BN, BN), lambda i: (0, 0)),
            pl.BlockSpec((TOP_K, KE), lambda i: (0, 0)),
        ],
        out_specs=[
            pl.BlockSpec((BN, KE), lambda i: (jnp.maximum(i - G, 0), 0)),
            pl.BlockSpec((BN, TOP_K), lambda i: (jnp.maximum(i - G, 0), 0)),
            pl.BlockSpec((BN, TOP_K), lambda i: (jnp.maximum(i - G, 0), 0)),
            pl.BlockSpec((BN, TOP_K), lambda i: (jnp.maximum(i - G, 0), 0)),
        ],
        out_shape=[
            jax.ShapeDtypeStruct((num_tokens, KE), jnp.int32),
            jax.ShapeDtypeStruct((num_tokens, TOP_K), jnp.float32),
            jax.ShapeDtypeStruct((num_tokens, TOP_K), jnp.int32),
            jax.ShapeDtypeStruct((num_tokens, TOP_K), jnp.int32),
        ],
        scratch_shapes=[
            pltpu.VMEM((G * TOP_K, BN), jnp.int32),
            pltpu.VMEM((G * TOP_K, BN), jnp.float32),
            pltpu.VMEM((G * TOP_K, BN), jnp.float32),
            pltpu.VMEM((G, KE), jnp.float32),
        ],
    )(x_flat, w_g, u_incl, jt)

    return (mask.reshape(num_tokens, TOP_K, N_EXP), pmask, idxo, rank)


# speedup vs baseline: 1.6059x; 1.1416x over previous
import functools
import math

import jax
import jax.numpy as jnp
from jax.experimental import pallas as pl
from jax.experimental.pallas import tpu as pltpu

TOP_K = 8
N_EXP = 64
EVAL_CAPACITY = 1.25
MIN_CAPACITY = 4

BN = 512


def _capacity(num_tokens: int) -> int:
    capacity = math.floor(TOP_K * EVAL_CAPACITY * num_tokens / N_EXP)
    capacity += capacity % 2
    capacity = max(capacity, MIN_CAPACITY)
    return int(capacity)


def _body(capacity, nblocks,
          x_ref, wg_ref, u_ref, jt_ref,
          mask_ref, pmask_ref, idxo_ref, rank_ref,
          idx_s, probs_s, rloc_s, hist_s):
    i = pl.program_id(0)
    G = nblocks
    KE = TOP_K * N_EXP

    @pl.when(i < G)
    def _phase1():
        xb = x_ref[...]
        wg = wg_ref[...]
        logitsT = jax.lax.dot_general(
            wg, xb, (((1,), (1,)), ((), ())),
            preferred_element_type=jnp.float32,
        )

        iota_sub = jax.lax.broadcasted_iota(jnp.int32, (N_EXP, BN), 0)
        work = logitsT
        idx_rows = []
        val_rows = []
        for _ in range(TOP_K):
            m = jnp.max(work, axis=0, keepdims=True)
            sel = jnp.where(work == m, iota_sub, N_EXP)
            ij = jnp.min(sel, axis=0, keepdims=True)
            idx_rows.append(ij)
            val_rows.append(m)
            work = jnp.where(iota_sub == ij, -jnp.inf, work)
        idxT = jnp.concatenate(idx_rows, axis=0)
        tvT = jnp.concatenate(val_rows, axis=0)

        e = jnp.exp(tvT - tvT[0:1, :])
        probsT = e / jnp.sum(e, axis=0, keepdims=True)

        ohT = jnp.concatenate(
            [(idxT[j : j + 1, :] == iota_sub).astype(jnp.float32)
             for j in range(TOP_K)],
            axis=0,
        )
        csumT = jax.lax.dot_general(
            ohT, u_ref[...], (((1,), (0,)), ((), ())),
            preferred_element_type=jnp.float32,
        )
        rloc_rows = []
        for j in range(TOP_K):
            sl = slice(j * N_EXP, (j + 1) * N_EXP)
            rloc_rows.append(
                jnp.sum(ohT[sl, :] * csumT[sl, :], axis=0, keepdims=True))
        rlocT = jnp.concatenate(rloc_rows, axis=0) - 1.0

        ones_row = jnp.ones((1, BN), jnp.float32)
        hist_row = jax.lax.dot_general(
            ones_row, ohT, (((1,), (1,)), ((), ())),
            preferred_element_type=jnp.float32,
        )

        idx_s[pl.ds(i * TOP_K, TOP_K), :] = idxT
        probs_s[pl.ds(i * TOP_K, TOP_K), :] = probsT
        rloc_s[pl.ds(i * TOP_K, TOP_K), :] = rlocT
        hist_s[pl.ds(i, 1), :] = hist_row

    @pl.when(i >= G)
    def _phase2():
        b = i - G
        idxT = idx_s[pl.ds(b * TOP_K, TOP_K), :]
        probsT = probs_s[pl.ds(b * TOP_K, TOP_K), :]
        rlocT = rloc_s[pl.ds(b * TOP_K, TOP_K), :]
        hist = hist_s[...]

        gi = jax.lax.broadcasted_iota(jnp.int32, (G, 2), 0)
        ci = jax.lax.broadcasted_iota(jnp.int32, (G, 2), 1)
        selm = jnp.where(ci == 0, (gi < b).astype(jnp.float32), 1.0)
        bt = jax.lax.dot_general(
            hist, selm, (((0,), (0,)), ((), ())),
            preferred_element_type=jnp.float32,
            precision=jax.lax.Precision.HIGHEST,
        )
        bexcl = bt[:, 0:1]
        tot = bt[:, 1:2]

        iota_sub = jax.lax.broadcasted_iota(jnp.int32, (N_EXP, BN), 0)
        acc = jnp.zeros((N_EXP, 1), jnp.float32)
        rank_rows = []
        keep_rows = []
        for j in range(TOP_K):
            sl = slice(j * N_EXP, (j + 1) * N_EXP)
            base_j = acc + bexcl[sl, :]
            mask_j = idxT[j : j + 1, :] == iota_sub
            contrib = jnp.sum(jnp.where(mask_j, base_j, 0.0),
                              axis=0, keepdims=True)
            rank_j = rlocT[j : j + 1, :] + contrib
            keep_j = (rank_j < float(capacity)).astype(jnp.float32)
            rank_rows.append(rank_j)
            keep_rows.append(keep_j)
            acc = acc + tot[sl, :]
        rankT = jnp.concatenate(rank_rows, axis=0)
        keepT = jnp.concatenate(keep_rows, axis=0)

        stack = jnp.concatenate(
            [idxT.astype(jnp.float32), probsT, rankT, keepT], axis=0)
        st = stack.T
        idx_tok = st[:, 0 : TOP_K]
        probs_tok = st[:, TOP_K : 2 * TOP_K]
        rank_tok = st[:, 2 * TOP_K : 3 * TOP_K]
        keep_tok = st[:, 3 * TOP_K : 4 * TOP_K]

        pmask_ref[...] = probs_tok * keep_tok
        idxo_ref[...] = idx_tok.astype(jnp.int32)
        rank_ref[...] = rank_tok.astype(jnp.int32)

        ce = jax.lax.dot_general(
            idx_tok, jt_ref[...], (((1,), (0,)), ((), ())),
            preferred_element_type=jnp.float32,
        )
        keep_exp = jax.lax.dot_general(
            keep_tok, jt_ref[...], (((1,), (0,)), ((), ())),
            preferred_element_type=jnp.float32,
        )
        erow = (jax.lax.broadcasted_iota(jnp.int32, (1, KE), 1)
                & (N_EXP - 1)).astype(jnp.float32)
        maskm = jnp.where(ce == erow, keep_exp, 0.0)
        mask_ref[...] = maskm.astype(jnp.int32)


def kernel(x, w_g):
    B, T, C = x.shape
    num_tokens = B * T
    x_flat = x.reshape(num_tokens, C)
    G = num_tokens // BN
    capacity = _capacity(num_tokens)
    KE = TOP_K * N_EXP

    r_i = jax.lax.broadcasted_iota(jnp.int32, (BN, BN), 0)
    c_i = jax.lax.broadcasted_iota(jnp.int32, (BN, BN), 1)
    u_incl = (r_i <= c_i).astype(jnp.float32)
    jr = jax.lax.broadcasted_iota(jnp.int32, (TOP_K, KE), 0)
    jc = jax.lax.broadcasted_iota(jnp.int32, (TOP_K, KE), 1)
    jt = (jr == jc // N_EXP).astype(jnp.float32)

    mask, pmask, idxo, rank = pl.pallas_call(
        functools.partial(_body, capacity, G),
        grid=(2 * G,),
        in_specs=[
            pl.BlockSpec((BN, C), lambda i: (jnp.minimum(i, G - 1), 0)),
            pl.BlockSpec((N_EXP, C), lambda i: (0, 0)),
            pl.BlockSpec((BN, BN), lambda i: (0, 0)),
            pl.BlockSpec((TOP_K, KE), lambda i: (0, 0)),
        ],
        out_specs=[
            pl.BlockSpec((BN, KE), lambda i: (jnp.maximum(i - G, 0), 0)),
            pl.BlockSpec((BN, TOP_K), lambda i: (jnp.maximum(i - G, 0), 0)),
            pl.BlockSpec((BN, TOP_K), lambda i: (jnp.maximum(i - G, 0), 0)),
            pl.BlockSpec((BN, TOP_K), lambda i: (jnp.maximum(i - G, 0), 0)),
        ],
        out_shape=[
            jax.ShapeDtypeStruct((num_tokens, KE), jnp.int32),
            jax.ShapeDtypeStruct((num_tokens, TOP_K), jnp.float32),
            jax.ShapeDtypeStruct((num_tokens, TOP_K), jnp.int32),
            jax.ShapeDtypeStruct((num_tokens, TOP_K), jnp.int32),
        ],
        scratch_shapes=[
            pltpu.VMEM((G * TOP_K, BN), jnp.int32),
            pltpu.VMEM((G * TOP_K, BN), jnp.float32),
            pltpu.VMEM((G * TOP_K, BN), jnp.float32),
            pltpu.VMEM((G, KE), jnp.float32),
        ],
    )(x_flat, w_g, u_incl, jt)

    return (mask.reshape(num_tokens, TOP_K, N_EXP), pmask, idxo, rank)


# speedup vs baseline: 1.7530x; 1.0916x over previous
import functools
import math

import jax
import jax.numpy as jnp
from jax.experimental import pallas as pl
from jax.experimental.pallas import tpu as pltpu

TOP_K = 8
N_EXP = 64
EVAL_CAPACITY = 1.25
MIN_CAPACITY = 4

BN = 1024


def _capacity(num_tokens: int) -> int:
    capacity = math.floor(TOP_K * EVAL_CAPACITY * num_tokens / N_EXP)
    capacity += capacity % 2
    capacity = max(capacity, MIN_CAPACITY)
    return int(capacity)


def _body(capacity, nblocks,
          x_ref, wg_ref, u_ref, jt_ref,
          mask_ref, pmask_ref, idxo_ref, rank_ref,
          idx_s, probs_s, rloc_s, hist_s):
    i = pl.program_id(0)
    G = nblocks
    KE = TOP_K * N_EXP

    @pl.when(i < G)
    def _phase1():
        xb = x_ref[...]
        wg = wg_ref[...]
        logitsT = jax.lax.dot_general(
            wg, xb, (((1,), (1,)), ((), ())),
            preferred_element_type=jnp.float32,
        )

        iota_sub = jax.lax.broadcasted_iota(jnp.int32, (N_EXP, BN), 0)
        work = logitsT
        idx_rows = []
        val_rows = []
        for _ in range(TOP_K):
            m = jnp.max(work, axis=0, keepdims=True)
            sel = jnp.where(work == m, iota_sub, N_EXP)
            ij = jnp.min(sel, axis=0, keepdims=True)
            idx_rows.append(ij)
            val_rows.append(m)
            work = jnp.where(iota_sub == ij, -jnp.inf, work)
        idxT = jnp.concatenate(idx_rows, axis=0)
        tvT = jnp.concatenate(val_rows, axis=0)

        e = jnp.exp(tvT - tvT[0:1, :])
        probsT = e / jnp.sum(e, axis=0, keepdims=True)

        ohT = jnp.concatenate(
            [(idxT[j : j + 1, :] == iota_sub).astype(jnp.float32)
             for j in range(TOP_K)],
            axis=0,
        )
        csumT = jax.lax.dot_general(
            ohT, u_ref[...], (((1,), (0,)), ((), ())),
            preferred_element_type=jnp.float32,
        )
        rloc_rows = []
        for j in range(TOP_K):
            sl = slice(j * N_EXP, (j + 1) * N_EXP)
            rloc_rows.append(
                jnp.sum(ohT[sl, :] * csumT[sl, :], axis=0, keepdims=True))
        rlocT = jnp.concatenate(rloc_rows, axis=0) - 1.0

        ones_row = jnp.ones((1, BN), jnp.float32)
        hist_row = jax.lax.dot_general(
            ones_row, ohT, (((1,), (1,)), ((), ())),
            preferred_element_type=jnp.float32,
        )

        idx_s[pl.ds(i * TOP_K, TOP_K), :] = idxT
        probs_s[pl.ds(i * TOP_K, TOP_K), :] = probsT
        rloc_s[pl.ds(i * TOP_K, TOP_K), :] = rlocT
        hist_s[pl.ds(i, 1), :] = hist_row

    @pl.when(i >= G)
    def _phase2():
        b = i - G
        idxT = idx_s[pl.ds(b * TOP_K, TOP_K), :]
        probsT = probs_s[pl.ds(b * TOP_K, TOP_K), :]
        rlocT = rloc_s[pl.ds(b * TOP_K, TOP_K), :]
        hist = hist_s[...]

        gi = jax.lax.broadcasted_iota(jnp.int32, (G, 2), 0)
        ci = jax.lax.broadcasted_iota(jnp.int32, (G, 2), 1)
        selm = jnp.where(ci == 0, (gi < b).astype(jnp.float32), 1.0)
        bt = jax.lax.dot_general(
            hist, selm, (((0,), (0,)), ((), ())),
            preferred_element_type=jnp.float32,
            precision=jax.lax.Precision.HIGHEST,
        )
        bexcl = bt[:, 0:1]
        tot = bt[:, 1:2]

        iota_sub = jax.lax.broadcasted_iota(jnp.int32, (N_EXP, BN), 0)
        acc = jnp.zeros((N_EXP, 1), jnp.float32)
        rank_rows = []
        keep_rows = []
        for j in range(TOP_K):
            sl = slice(j * N_EXP, (j + 1) * N_EXP)
            base_j = acc + bexcl[sl, :]
            mask_j = idxT[j : j + 1, :] == iota_sub
            contrib = jnp.sum(jnp.where(mask_j, base_j, 0.0),
                              axis=0, keepdims=True)
            rank_j = rlocT[j : j + 1, :] + contrib
            keep_j = (rank_j < float(capacity)).astype(jnp.float32)
            rank_rows.append(rank_j)
            keep_rows.append(keep_j)
            acc = acc + tot[sl, :]
        rankT = jnp.concatenate(rank_rows, axis=0)
        keepT = jnp.concatenate(keep_rows, axis=0)

        stack = jnp.concatenate(
            [idxT.astype(jnp.float32), probsT, rankT, keepT], axis=0)
        st = stack.T
        idx_tok = st[:, 0 : TOP_K]
        probs_tok = st[:, TOP_K : 2 * TOP_K]
        rank_tok = st[:, 2 * TOP_K : 3 * TOP_K]
        keep_tok = st[:, 3 * TOP_K : 4 * TOP_K]

        pmask_ref[...] = probs_tok * keep_tok
        idxo_ref[...] = idx_tok.astype(jnp.int32)
        rank_ref[...] = rank_tok.astype(jnp.int32)

        ce = jax.lax.dot_general(
            idx_tok, jt_ref[...], (((1,), (0,)), ((), ())),
            preferred_element_type=jnp.float32,
        )
        keep_exp = jax.lax.dot_general(
            keep_tok, jt_ref[...], (((1,), (0,)), ((), ())),
            preferred_element_type=jnp.float32,
        )
        erow = (jax.lax.broadcasted_iota(jnp.int32, (1, KE), 1)
                & (N_EXP - 1)).astype(jnp.float32)
        maskm = jnp.where(ce == erow, keep_exp, 0.0)
        mask_ref[...] = maskm.astype(jnp.int32)


def kernel(x, w_g):
    B, T, C = x.shape
    num_tokens = B * T
    x_flat = x.reshape(num_tokens, C)
    G = num_tokens // BN
    capacity = _capacity(num_tokens)
    KE = TOP_K * N_EXP

    r_i = jax.lax.broadcasted_iota(jnp.int32, (BN, BN), 0)
    c_i = jax.lax.broadcasted_iota(jnp.int32, (BN, BN), 1)
    u_incl = (r_i <= c_i).astype(jnp.float32)
    jr = jax.lax.broadcasted_iota(jnp.int32, (TOP_K, KE), 0)
    jc = jax.lax.broadcasted_iota(jnp.int32, (TOP_K, KE), 1)
    jt = (jr == jc // N_EXP).astype(jnp.float32)

    mask, pmask, idxo, rank = pl.pallas_call(
        functools.partial(_body, capacity, G),
        grid=(2 * G,),
        in_specs=[
            pl.BlockSpec((BN, C), lambda i: (jnp.minimum(i, G - 1), 0)),
            pl.BlockSpec((N_EXP, C), lambda i: (0, 0)),
            pl.BlockSpec((BN, BN), lambda i: (0, 0)),
            pl.BlockSpec((TOP_K, KE), lambda i: (0, 0)),
        ],
        out_specs=[
            pl.BlockSpec((BN, KE), lambda i: (jnp.maximum(i - G, 0), 0)),
            pl.BlockSpec((BN, TOP_K), lambda i: (jnp.maximum(i - G, 0), 0)),
            pl.BlockSpec((BN, TOP_K), lambda i: (jnp.maximum(i - G, 0), 0)),
            pl.BlockSpec((BN, TOP_K), lambda i: (jnp.maximum(i - G, 0), 0)),
        ],
        out_shape=[
            jax.ShapeDtypeStruct((num_tokens, KE), jnp.int32),
            jax.ShapeDtypeStruct((num_tokens, TOP_K), jnp.float32),
            jax.ShapeDtypeStruct((num_tokens, TOP_K), jnp.int32),
            jax.ShapeDtypeStruct((num_tokens, TOP_K), jnp.int32),
        ],
        scratch_shapes=[
            pltpu.VMEM((G * TOP_K, BN), jnp.int32),
            pltpu.VMEM((G * TOP_K, BN), jnp.float32),
            pltpu.VMEM((G * TOP_K, BN), jnp.float32),
            pltpu.VMEM((G, KE), jnp.float32),
        ],
    )(x_flat, w_g, u_incl, jt)

    return (mask.reshape(num_tokens, TOP_K, N_EXP), pmask, idxo, rank)
